# baseline (speedup 1.0000x reference)
import functools

import jax
import jax.numpy as jnp
from jax import lax
from jax.experimental import pallas as pl
from jax.experimental.pallas import tpu as pltpu
from jax.experimental.pallas import tpu_sc as plsc

MAX_N = 118
D = 128
BATCH = 16384
SEQ = 200
N = BATCH * SEQ
NC = 2
NS = 16
NW = NC * NS
BPW = N // NW
SUB = 128
CHUNK = 256
NSUB = CHUNK // SUB
ITERS = BPW // CHUNK
IDXR_PW = BPW // SUB
IDX_PAD = 64


def _sc_gather(table, idx2d):
  mesh = plsc.VectorSubcoreMesh(core_axis_name="c", subcore_axis_name="s")

  @functools.partial(
      pl.kernel,
      mesh=mesh,
      out_type=jax.ShapeDtypeStruct((N, D), jnp.float32),
      scratch_types=[
          pltpu.VMEM((2, NSUB, SUB), jnp.int32),
          pltpu.VMEM((2, CHUNK, D), jnp.float32),
          pltpu.VMEM_SHARED((NS * 128, D), jnp.float32),
          pltpu.SemaphoreType.DMA,
          pltpu.SemaphoreType.DMA,
          pltpu.SemaphoreType.DMA,
          pltpu.SemaphoreType.DMA,
          pltpu.SemaphoreType.DMA,
          pltpu.SemaphoreType.DMA,
      ],
  )
  def body(table_hbm, idx_hbm, out_hbm, idx_v, rows_v, tab_sp,
           si0, si1, sg0, sg1, so0, so1):
    sid = lax.axis_index("s")
    wid = sid * NC + lax.axis_index("c")
    row0 = wid * BPW
    irow0 = wid * IDXR_PW
    s_idx = (si0, si1)
    s_gat = (sg0, sg1)
    s_out = (so0, so1)

    pltpu.sync_copy(table_hbm.at[pl.ds(wid * 128, 128)],
                    tab_sp.at[pl.ds(sid * 128, 128)])
    plsc.subcore_barrier()

    def idx_cp(i, b):
      return pltpu.make_async_copy(
          idx_hbm.at[pl.ds(irow0 + i * NSUB, NSUB)], idx_v.at[b], s_idx[b])

    def gather_cp(b, j):
      return pltpu.make_async_copy(
          tab_sp.at[idx_v.at[b].at[j]],
          rows_v.at[b].at[pl.ds(j * SUB, SUB)],
          s_gat[b])

    def out_cp(i, b):
      return pltpu.make_async_copy(
          rows_v.at[b], out_hbm.at[pl.ds(row0 + i * CHUNK, CHUNK)], s_out[b])

    idx_cp(0, 0).start()
    idx_cp(1, 1).start()

    def step(k, carry):
      g = 2 * k
      for b in range(2):
        i = g + b
        idx_cp(i, b).wait()

        @pl.when(k >= 1)
        def _wait_out():
          out_cp(i - 2, b).wait()

        for j in range(NSUB):
          gather_cp(b, j).start()
        for j in range(NSUB):
          gather_cp(b, j).wait()
        idx_cp(i + 2, b).start()

        out_cp(i, b).start()
      return carry

    lax.fori_loop(0, ITERS // 2, step, 0)

    for b in range(2):
      out_cp(ITERS - 2 + b, b).wait()
      idx_cp(0, b).wait()

  return body(table, idx2d)


def kernel(inputs, embedding):
  table = jnp.tile(
      jnp.zeros((128, D), jnp.float32).at[1:MAX_N + 1].set(embedding),
      (NW, 1))
  r = jnp.arange(N // SUB + IDX_PAD, dtype=jnp.int32)
  off = (jnp.minimum(r // (NC * IDXR_PW), NS - 1) * 128)[:, None]
  idx2d = jnp.concatenate(
      [inputs.reshape(N // SUB, SUB),
       jnp.zeros((IDX_PAD, SUB), jnp.int32)], axis=0) + off
  out = _sc_gather(table, idx2d)
  return out.reshape(BATCH, SEQ, D)


# speedup vs baseline: 1.0182x; 1.0182x over previous
import functools

import jax
import jax.numpy as jnp
from jax import lax
from jax.experimental import pallas as pl
from jax.experimental.pallas import tpu as pltpu
from jax.experimental.pallas import tpu_sc as plsc

MAX_N = 118
D = 128
BATCH = 16384
SEQ = 200
N = BATCH * SEQ
NC = 2
NS = 16
NW = NC * NS
BPW = N // NW
SUB = 128
CHUNK = 256
NSUB = CHUNK // SUB
ITERS = BPW // CHUNK
IDXR_PW = BPW // SUB
IDX_PAD = 64


def _sc_gather(table, idx2d):
  mesh = plsc.VectorSubcoreMesh(core_axis_name="c", subcore_axis_name="s")

  @functools.partial(
      pl.kernel,
      mesh=mesh,
      out_type=jax.ShapeDtypeStruct((N, D), jnp.float32),
      scratch_types=[
          pltpu.VMEM((2, NSUB, SUB), jnp.int32),
          pltpu.VMEM((2, CHUNK, D), jnp.float32),
          pltpu.VMEM_SHARED((NS * 128, D), jnp.float32),
          pltpu.SemaphoreType.DMA,
          pltpu.SemaphoreType.DMA,
          pltpu.SemaphoreType.DMA,
          pltpu.SemaphoreType.DMA,
          pltpu.SemaphoreType.DMA,
          pltpu.SemaphoreType.DMA,
      ],
  )
  def body(table_hbm, idx_hbm, out_hbm, idx_v, rows_v, tab_sp,
           si0, si1, sg0, sg1, so0, so1):
    sid = lax.axis_index("s")
    wid = sid * NC + lax.axis_index("c")
    row0 = wid * BPW
    irow0 = wid * IDXR_PW
    s_idx = (si0, si1)
    s_gat = (sg0, sg1)
    s_out = (so0, so1)

    pltpu.sync_copy(table_hbm, tab_sp.at[pl.ds(sid * 128, 128)])
    plsc.subcore_barrier()

    def idx_cp(i, b):
      return pltpu.make_async_copy(
          idx_hbm.at[pl.ds(irow0 + i * NSUB, NSUB)], idx_v.at[b], s_idx[b])

    def gather_cp(b, j):
      return pltpu.make_async_copy(
          tab_sp.at[idx_v.at[b].at[j]],
          rows_v.at[b].at[pl.ds(j * SUB, SUB)],
          s_gat[b])

    def out_cp(i, b):
      return pltpu.make_async_copy(
          rows_v.at[b], out_hbm.at[pl.ds(row0 + i * CHUNK, CHUNK)], s_out[b])

    idx_cp(0, 0).start()
    idx_cp(1, 1).start()

    def step(k, carry):
      g = 2 * k
      for b in range(2):
        i = g + b
        idx_cp(i, b).wait()
        off = sid * 128
        for j in range(NSUB):
          for l in range(SUB // 16):
            sl = idx_v.at[b].at[j]
            sl[pl.ds(l * 16, 16)] = sl[pl.ds(l * 16, 16)] + off

        @pl.when(k >= 1)
        def _wait_out():
          out_cp(i - 2, b).wait()

        for j in range(NSUB):
          gather_cp(b, j).start()
        for j in range(NSUB):
          gather_cp(b, j).wait()

        @pl.when(i + 2 < ITERS)
        def _prefetch_idx():
          idx_cp(i + 2, b).start()

        out_cp(i, b).start()
      return carry

    lax.fori_loop(0, ITERS // 2, step, 0)

    for b in range(2):
      out_cp(ITERS - 2 + b, b).wait()

  return body(table, idx2d)


def kernel(inputs, embedding):
  table = jnp.zeros((128, D), jnp.float32).at[1:MAX_N + 1].set(embedding)
  idx2d = inputs.reshape(N // SUB, SUB)
  out = _sc_gather(table, idx2d)
  return out.reshape(BATCH, SEQ, D)
